# baseline (speedup 1.0000x reference)
import functools

import jax
import jax.numpy as jnp
from jax.experimental import pallas as pl
from jax.experimental.pallas import tpu as pltpu


_VMEM_LIMIT_BYTES = 48 * 1024 * 1024


def _pool_linear_kernel(x_ref, w_ref, b_ref, o_ref, *, inv_hw, n_label):
    pooled = jnp.sum(x_ref[...], axis=0) * inv_hw
    y = jnp.dot(pooled, w_ref[...], preferred_element_type=jnp.float32)
    y = (y + b_ref[...]).astype(o_ref.dtype)
    o_ref[...] = y.T[:n_label, :]


def _choose_tb(batch, hw, c, itemsize):
    budget = 16 * 1024 * 1024
    cap = max(128, budget // (hw * c * itemsize))
    for tb in (256, 128):
        if tb <= cap and batch % tb == 0 and batch // tb >= 2:
            return tb
    return batch


def kernel(x, weight_t, bias2):
    B, C, H, W = x.shape
    HW = H * W
    Lp = weight_t.shape[1]
    n_label = 1000

    x3 = x.transpose(2, 3, 0, 1).reshape(HW, B, C)

    tb = _choose_tb(B, HW, C, x.dtype.itemsize)
    grid = (B // tb,)

    cost = pl.CostEstimate(
        flops=int(B * C * HW + 2 * B * C * Lp),
        transcendentals=0,
        bytes_accessed=int(x.dtype.itemsize * B * C * HW
                           + 4 * (C * Lp + Lp + B * Lp)))

    out = pl.pallas_call(
        functools.partial(_pool_linear_kernel, inv_hw=float(1.0 / HW),
                          n_label=n_label),
        out_shape=jax.ShapeDtypeStruct((n_label, B), jnp.float32),
        grid=grid,
        in_specs=[
            pl.BlockSpec((HW, tb, C), lambda i: (0, i, 0)),
            pl.BlockSpec((C, Lp), lambda i: (0, 0)),
            pl.BlockSpec((1, Lp), lambda i: (0, 0)),
        ],
        out_specs=pl.BlockSpec((n_label, tb), lambda i: (0, i)),
        compiler_params=pltpu.CompilerParams(
            dimension_semantics=("parallel",),
            vmem_limit_bytes=_VMEM_LIMIT_BYTES),
        cost_estimate=cost,
    )(x3, weight_t, bias2)

    return out.T
